# Initial kernel scaffold; baseline (speedup 1.0000x reference)
import jax
import jax.numpy as jnp
from jax.experimental import pallas as pl


def kernel(A_on, P_on, N_on, A_in, P_in, N_in, A_prox, P_prox, N_prox, A_X, P_X, N_X, W_on_0, b_on_0, W_on_1, b_on_1, W_in_0, b_in_0, W_in_1, b_in_1, W_prox_0, b_prox_0, W_prox_1, b_prox_1, fc1_W, fc1_b, fc2_W, fc2_b):
    raise NotImplementedError("write your pallas kernel here")


# speedup vs baseline: 1.8719x; 1.8719x over previous
import jax
import jax.numpy as jnp
from jax.experimental import pallas as pl

F32 = jnp.float32
B, N = 32, 64
G = 8
NB = 3 * B
FC1_BLK = 512
FC2_BLK = 512


def _gcn_kernel(x_ref, aon_ref, ain_ref, apr_ref,
                won0_ref, bon0_ref, won1_ref, bon1_ref,
                win0_ref, bin0_ref, win1_ref, bin1_ref,
                wpr0_ref, bpr0_ref, wpr1_ref, bpr1_ref,
                out_ref):
    x2 = x_ref[...].reshape(G * N, 256)

    def path(a_ref, w0_ref, b0_ref, w1_ref, b1_ref, off):
        a = a_ref[...]
        s0 = jnp.dot(x2, w0_ref[...], preferred_element_type=F32)
        s0 = s0.reshape(G, N, 128)
        h0 = jax.lax.dot_general(a, s0, (((2,), (1,)), ((0,), (0,))),
                                 preferred_element_type=F32)
        h0 = jnp.maximum(h0 + b0_ref[...], 0.0)
        s1 = jnp.dot(h0.reshape(G * N, 128), w1_ref[...],
                     preferred_element_type=F32)
        s1 = s1.reshape(G, N, 64)
        h1 = jax.lax.dot_general(a, s1, (((2,), (1,)), ((0,), (0,))),
                                 preferred_element_type=F32)
        out_ref[:, :, off:off + 64] = jnp.maximum(h1 + b1_ref[...], 0.0)

    path(aon_ref, won0_ref, bon0_ref, won1_ref, bon1_ref, 0)
    path(ain_ref, win0_ref, bin0_ref, win1_ref, bin1_ref, 64)
    path(apr_ref, wpr0_ref, bpr0_ref, wpr1_ref, bpr1_ref, 128)


def _fc_kernel(lhs_ref, w_ref, b_ref, out_ref):
    out_ref[...] = jnp.maximum(
        jnp.dot(lhs_ref[...], w_ref[...], preferred_element_type=F32)
        + b_ref[...], 0.0)


def _gcn(X, Aon, Ain, Apr, weights):
    in_specs = [
        pl.BlockSpec((G, N, 256), lambda i: (i, 0, 0)),
        pl.BlockSpec((G, N, N), lambda i: (i, 0, 0)),
        pl.BlockSpec((G, N, N), lambda i: (i, 0, 0)),
        pl.BlockSpec((G, N, N), lambda i: (i, 0, 0)),
    ]
    for w in weights:
        in_specs.append(pl.BlockSpec(w.shape, lambda i: (0,) * w.ndim))
    return pl.pallas_call(
        _gcn_kernel,
        grid=(NB // G,),
        in_specs=in_specs,
        out_specs=pl.BlockSpec((G, N, 192), lambda i: (i, 0, 0)),
        out_shape=jax.ShapeDtypeStruct((NB, N, 192), F32),
    )(X, Aon, Ain, Apr, *weights)


def _fc(lhs, W, b, blk):
    K, M = W.shape
    rows = lhs.shape[0]
    return pl.pallas_call(
        _fc_kernel,
        grid=(M // blk,),
        in_specs=[
            pl.BlockSpec((rows, K), lambda i: (0, 0)),
            pl.BlockSpec((K, blk), lambda i: (0, i)),
            pl.BlockSpec((1, blk), lambda i: (0, i)),
        ],
        out_specs=pl.BlockSpec((rows, blk), lambda i: (0, i)),
        out_shape=jax.ShapeDtypeStruct((rows, M), F32),
    )(lhs, W, b.reshape(1, -1))


def kernel(A_on, P_on, N_on, A_in, P_in, N_in, A_prox, P_prox, N_prox,
           A_X, P_X, N_X,
           W_on_0, b_on_0, W_on_1, b_on_1,
           W_in_0, b_in_0, W_in_1, b_in_1,
           W_prox_0, b_prox_0, W_prox_1, b_prox_1,
           fc1_W, fc1_b, fc2_W, fc2_b):
    X = jnp.concatenate([A_X, P_X, N_X], axis=0)
    Aon = jnp.concatenate([A_on, P_on, N_on], axis=0)
    Ain = jnp.concatenate([A_in, P_in, N_in], axis=0)
    Apr = jnp.concatenate([A_prox, P_prox, N_prox], axis=0)
    r = lambda b: b.reshape(1, -1)
    weights = (W_on_0, r(b_on_0), W_on_1, r(b_on_1),
               W_in_0, r(b_in_0), W_in_1, r(b_in_1),
               W_prox_0, r(b_prox_0), W_prox_1, r(b_prox_1))
    cat = _gcn(X, Aon, Ain, Apr, weights).reshape(NB, N * 192)
    h = _fc(cat, fc1_W, fc1_b, FC1_BLK)
    out = _fc(h, fc2_W, fc2_b, FC2_BLK)
    return (out[0:B], out[B:2 * B], out[2 * B:3 * B])


# speedup vs baseline: 2.4380x; 1.3024x over previous
import jax
import jax.numpy as jnp
from jax.experimental import pallas as pl
from jax.experimental.pallas import tpu as pltpu

F32 = jnp.float32
B, N = 32, 64
NB = 3 * B
D_CAT = N * 192
D_H = 4096
D_OUT = 2048
FC1_BLK = 256
FC2_BLK = 128
FC1_STEPS = D_H // FC1_BLK
FC2_STEPS = D_OUT // FC2_BLK


def _fused_kernel(x0_ref, x1_ref, x2_ref,
                  aon0_ref, aon1_ref, aon2_ref,
                  ain0_ref, ain1_ref, ain2_ref,
                  apr0_ref, apr1_ref, apr2_ref,
                  won0_ref, bon0_ref, won1_ref, bon1_ref,
                  win0_ref, bin0_ref, win1_ref, bin1_ref,
                  wpr0_ref, bpr0_ref, wpr1_ref, bpr1_ref,
                  fc1w_ref, fc1b_ref, fc2w_ref, fc2b_ref,
                  out_ref, cat3_scr, cat2_scr, h_scr):
    step = pl.program_id(0)

    @pl.when(step == 0)
    def _gcn():
        xs = (x0_ref, x1_ref, x2_ref)
        paths = (
            ((aon0_ref, aon1_ref, aon2_ref), won0_ref, bon0_ref, won1_ref, bon1_ref, 0),
            ((ain0_ref, ain1_ref, ain2_ref), win0_ref, bin0_ref, win1_ref, bin1_ref, 64),
            ((apr0_ref, apr1_ref, apr2_ref), wpr0_ref, bpr0_ref, wpr1_ref, bpr1_ref, 128),
        )
        for m in range(3):
            x2 = xs[m][...].reshape(B * N, 256)
            for a_refs, w0_ref, b0_ref, w1_ref, b1_ref, off in paths:
                a = a_refs[m][...]
                s0 = jnp.dot(x2, w0_ref[...], preferred_element_type=F32)
                h0 = jax.lax.dot_general(
                    a, s0.reshape(B, N, 128), (((2,), (1,)), ((0,), (0,))),
                    preferred_element_type=F32)
                h0 = jnp.maximum(h0 + b0_ref[...], 0.0)
                s1 = jnp.dot(h0.reshape(B * N, 128), w1_ref[...],
                             preferred_element_type=F32)
                h1 = jax.lax.dot_general(
                    a, s1.reshape(B, N, 64), (((2,), (1,)), ((0,), (0,))),
                    preferred_element_type=F32)
                cat3_scr[m * B:(m + 1) * B, :, off:off + 64] = \
                    jnp.maximum(h1 + b1_ref[...], 0.0)
        cat2_scr[...] = cat3_scr[...].reshape(NB, D_CAT)

    @pl.when(step < FC1_STEPS)
    def _fc1():
        blk = jnp.dot(cat2_scr[...], fc1w_ref[...],
                      preferred_element_type=F32)
        h_scr[:, pl.ds(step * FC1_BLK, FC1_BLK)] = \
            jnp.maximum(blk + fc1b_ref[...], 0.0)

    @pl.when(step >= FC1_STEPS)
    def _fc2():
        blk = jnp.dot(h_scr[...], fc2w_ref[...],
                      preferred_element_type=F32)
        out_ref[...] = jnp.maximum(blk + fc2b_ref[...], 0.0)


def kernel(A_on, P_on, N_on, A_in, P_in, N_in, A_prox, P_prox, N_prox,
           A_X, P_X, N_X,
           W_on_0, b_on_0, W_on_1, b_on_1,
           W_in_0, b_in_0, W_in_1, b_in_1,
           W_prox_0, b_prox_0, W_prox_1, b_prox_1,
           fc1_W, fc1_b, fc2_W, fc2_b):
    r = lambda b: b.reshape(1, -1)
    const = lambda arr: pl.BlockSpec(arr.shape, lambda i: (0,) * arr.ndim)
    gcn_inputs = (A_X, P_X, N_X, A_on, P_on, N_on, A_in, P_in, N_in,
                  A_prox, P_prox, N_prox,
                  W_on_0, r(b_on_0), W_on_1, r(b_on_1),
                  W_in_0, r(b_in_0), W_in_1, r(b_in_1),
                  W_prox_0, r(b_prox_0), W_prox_1, r(b_prox_1))
    in_specs = [const(a) for a in gcn_inputs]
    in_specs += [
        pl.BlockSpec((D_CAT, FC1_BLK),
                     lambda i: (0, jnp.minimum(i, FC1_STEPS - 1))),
        pl.BlockSpec((1, FC1_BLK),
                     lambda i: (0, jnp.minimum(i, FC1_STEPS - 1))),
        pl.BlockSpec((D_H, FC2_BLK),
                     lambda i: (0, jnp.maximum(i - FC1_STEPS, 0))),
        pl.BlockSpec((1, FC2_BLK),
                     lambda i: (0, jnp.maximum(i - FC1_STEPS, 0))),
    ]
    out = pl.pallas_call(
        _fused_kernel,
        grid=(FC1_STEPS + FC2_STEPS,),
        in_specs=in_specs,
        out_specs=pl.BlockSpec((NB, FC2_BLK),
                               lambda i: (0, jnp.maximum(i - FC1_STEPS, 0))),
        out_shape=jax.ShapeDtypeStruct((NB, D_OUT), F32),
        scratch_shapes=[
            pltpu.VMEM((NB, N, 192), F32),
            pltpu.VMEM((NB, D_CAT), F32),
            pltpu.VMEM((NB, D_H), F32),
        ],
    )(*gcn_inputs, fc1_W, r(fc1_b), fc2_W, r(fc2_b))
    return (out[0:B], out[B:2 * B], out[2 * B:3 * B])


# speedup vs baseline: 2.5679x; 1.0533x over previous
import jax
import jax.numpy as jnp
from jax.experimental import pallas as pl
from jax.experimental.pallas import tpu as pltpu

F32 = jnp.float32
B, N = 32, 64
NB = 3 * B
D_CAT = N * 192
D_H = 4096
D_OUT = 2048
FC1_BLK = 256
FC2_BLK = 128
FC1_CHUNKS = D_H // FC1_BLK
FC2_STEPS = D_OUT // FC2_BLK
LAG = 2
FC2_START = FC1_CHUNKS + LAG
NSTEPS = FC2_START + FC2_STEPS


def _fused_kernel(x0_ref, x1_ref, x2_ref,
                  aon0_ref, aon1_ref, aon2_ref,
                  ain0_ref, ain1_ref, ain2_ref,
                  apr0_ref, apr1_ref, apr2_ref,
                  won0_ref, bon0_ref, won1_ref, bon1_ref,
                  win0_ref, bin0_ref, win1_ref, bin1_ref,
                  wpr0_ref, bpr0_ref, wpr1_ref, bpr1_ref,
                  fc1w_hbm, fc1b_ref, fc2w_ref, fc2b_ref,
                  out_ref, cat3_scr, cat2_scr, h_scr, w_buf, w_sem):
    step = pl.program_id(0)

    def w_copy(c, nb):
        return pltpu.make_async_copy(
            fc1w_hbm.at[:, pl.ds(c * FC1_BLK, FC1_BLK)],
            w_buf.at[nb], w_sem.at[nb])

    @pl.when(step == 0)
    def _kick_first():
        w_copy(0, 0).start()
        w_copy(1, 1).start()

    @pl.when(step < 3)
    def _gcn():
        xs = (x0_ref, x1_ref, x2_ref)
        paths = (
            ((aon0_ref, aon1_ref, aon2_ref), won0_ref, bon0_ref, won1_ref, bon1_ref, 0),
            ((ain0_ref, ain1_ref, ain2_ref), win0_ref, bin0_ref, win1_ref, bin1_ref, 64),
            ((apr0_ref, apr1_ref, apr2_ref), wpr0_ref, bpr0_ref, wpr1_ref, bpr1_ref, 128),
        )
        for m in range(3):
            @pl.when(step == m)
            def _member():
                x2 = xs[m][...].reshape(B * N, 256)
                for a_refs, w0_ref, b0_ref, w1_ref, b1_ref, off in paths:
                    a = a_refs[m][...]
                    s0 = jnp.dot(x2, w0_ref[...], preferred_element_type=F32)
                    h0 = jax.lax.dot_general(
                        a, s0.reshape(B, N, 128), (((2,), (1,)), ((0,), (0,))),
                        preferred_element_type=F32)
                    h0 = jnp.maximum(h0 + b0_ref[...], 0.0)
                    s1 = jnp.dot(h0.reshape(B * N, 128), w1_ref[...],
                                 preferred_element_type=F32)
                    h1 = jax.lax.dot_general(
                        a, s1.reshape(B, N, 64), (((2,), (1,)), ((0,), (0,))),
                        preferred_element_type=F32)
                    cat3_scr[m * B:(m + 1) * B, :, off:off + 64] = \
                        jnp.maximum(h1 + b1_ref[...], 0.0)

    @pl.when(step == 2)
    def _flatten():
        cat2_scr[...] = cat3_scr[...].reshape(NB, D_CAT)

    @pl.when((step >= LAG) & (step < FC2_START))
    def _fc1():
        c = step - LAG
        nb = jax.lax.rem(c, 2)
        w_copy(c, nb).wait()
        blk = jnp.dot(cat2_scr[...], w_buf[nb], preferred_element_type=F32)
        h_scr[:, pl.ds(c * FC1_BLK, FC1_BLK)] = \
            jnp.maximum(blk + fc1b_ref[...], 0.0)

    @pl.when((step >= LAG) & (step < FC1_CHUNKS))
    def _kick_next():
        w_copy(step, jax.lax.rem(step, 2)).start()

    @pl.when(step >= FC2_START)
    def _fc2():
        blk = jnp.dot(h_scr[...], fc2w_ref[...], preferred_element_type=F32)
        out_ref[...] = jnp.maximum(blk + fc2b_ref[...], 0.0)


def kernel(A_on, P_on, N_on, A_in, P_in, N_in, A_prox, P_prox, N_prox,
           A_X, P_X, N_X,
           W_on_0, b_on_0, W_on_1, b_on_1,
           W_in_0, b_in_0, W_in_1, b_in_1,
           W_prox_0, b_prox_0, W_prox_1, b_prox_1,
           fc1_W, fc1_b, fc2_W, fc2_b):
    r = lambda b: b.reshape(1, -1)
    const = lambda arr: pl.BlockSpec(arr.shape, lambda i: (0,) * arr.ndim)
    gcn_inputs = (A_X, P_X, N_X, A_on, P_on, N_on, A_in, P_in, N_in,
                  A_prox, P_prox, N_prox,
                  W_on_0, r(b_on_0), W_on_1, r(b_on_1),
                  W_in_0, r(b_in_0), W_in_1, r(b_in_1),
                  W_prox_0, r(b_prox_0), W_prox_1, r(b_prox_1))
    in_specs = [const(a) for a in gcn_inputs]
    in_specs += [
        pl.BlockSpec(memory_space=pltpu.MemorySpace.HBM),
        pl.BlockSpec((1, FC1_BLK),
                     lambda i: (0, jnp.clip(i - LAG, 0, FC1_CHUNKS - 1))),
        pl.BlockSpec((D_H, FC2_BLK),
                     lambda i: (0, jnp.maximum(i - FC2_START, 0))),
        pl.BlockSpec((1, FC2_BLK),
                     lambda i: (0, jnp.maximum(i - FC2_START, 0))),
    ]
    out = pl.pallas_call(
        _fused_kernel,
        grid=(NSTEPS,),
        in_specs=in_specs,
        out_specs=pl.BlockSpec((NB, FC2_BLK),
                               lambda i: (0, jnp.maximum(i - FC2_START, 0))),
        out_shape=jax.ShapeDtypeStruct((NB, D_OUT), F32),
        scratch_shapes=[
            pltpu.VMEM((NB, N, 192), F32),
            pltpu.VMEM((NB, D_CAT), F32),
            pltpu.VMEM((NB, D_H), F32),
            pltpu.VMEM((2, D_CAT, FC1_BLK), F32),
            pltpu.SemaphoreType.DMA((2,)),
        ],
    )(*gcn_inputs, fc1_W, r(fc1_b), fc2_W, r(fc2_b))
    return (out[0:B], out[B:2 * B], out[2 * B:3 * B])
